# baseline (speedup 1.0000x reference)
import numpy as np
import jax
import jax.numpy as jnp
from jax.experimental import pallas as pl
from jax.experimental.pallas import tpu as pltpu

_N = 100000
_B = 2000
_NB = _N // _B
_G = 64
_BIG = np.int32(2**31 - 1)


def _threefry_bits(j):
    k0 = jnp.uint32(0)
    k1 = jnp.uint32(1234)
    ks2 = k0 ^ k1 ^ jnp.uint32(0x1BD11BDA)
    ks = [k0, k1, ks2]
    rots = [[13, 15, 26, 6], [17, 29, 16, 24]]
    x0 = jnp.zeros_like(j) + ks[0]
    x1 = j + ks[1]
    for i in range(5):
        for r in rots[i % 2]:
            x0 = x0 + x1
            x1 = (x1 << r) | (x1 >> (32 - r))
            x1 = x1 ^ x0
        x0 = x0 + ks[(i + 1) % 3]
        x1 = x1 + ks[(i + 2) % 3] + jnp.uint32(i + 1)
    return x0 ^ x1


def _gumbel_from_bits(bits):
    tiny = jnp.float32(np.finfo(np.float32).tiny)
    fb = (bits >> 9) | jnp.uint32(0x3F800000)
    f = jax.lax.bitcast_convert_type(fb, jnp.float32) - jnp.float32(1.0)
    u = jnp.maximum(tiny, f * (jnp.float32(1.0) - tiny) + tiny)
    return -jnp.log(-jnp.log(u))


def _onehot(b):
    gcol = 63 - jax.lax.broadcasted_iota(jnp.int32, (_G, 1), 0)
    return b == gcol


def _body(xb_ref, bt_ref, w0_ref, b0_ref, w1_ref, b1_ref, wf_ref,
          bf_ref, p_ref, act_ref, a_ref,
          gscr, escr, ssum, smax, sidx, spwin, scnt):
    ph = pl.program_id(0)
    step = pl.program_id(1)
    i = step * _B + jax.lax.broadcasted_iota(jnp.int32, (1, _B), 1)
    b = bt_ref[pl.ds(step, 1), :]

    @pl.when(ph == 0)
    def _phase_a():
        @pl.when(step == 0)
        def _init():
            ssum[...] = jnp.zeros_like(ssum)
            smax[...] = jnp.full_like(smax, -jnp.inf)
            sidx[...] = jnp.full_like(sidx, _BIG)
            spwin[...] = jnp.zeros_like(spwin)
            scnt[...] = jnp.zeros_like(scnt)
            for t in range(0, _NB, 8):
                rows = min(8, _NB - t)
                b8 = bt_ref[pl.ds(t, rows), :]
                i8 = (t * _B
                      + jax.lax.broadcasted_iota(jnp.int32, (rows, _B), 0) * _B
                      + jax.lax.broadcasted_iota(jnp.int32, (rows, _B), 1))
                j8 = ((63 - b8) * _N + i8).astype(jnp.uint32)
                gscr[pl.ds(t, rows), :] = _gumbel_from_bits(_threefry_bits(j8))

        xb = xb_ref[...]
        h = jax.lax.dot_general(xb, w0_ref[...], (((1,), (0,)), ((), ())),
                                preferred_element_type=jnp.float32)
        h = jnp.maximum(h + b0_ref[...], jnp.float32(0.0))
        h = jax.lax.dot_general(h, w1_ref[...], (((1,), (0,)), ((), ())),
                                preferred_element_type=jnp.float32)
        h = jnp.maximum(h + b1_ref[...], jnp.float32(0.0))
        logits = jax.lax.dot_general(wf_ref[...], h, (((1,), (1,)), ((), ())),
                                     preferred_element_type=jnp.float32)
        e = jnp.exp(logits + bf_ref[...])
        escr[pl.ds(step, 1), :] = e

        mf = _onehot(b).astype(jnp.float32)
        ssum[...] += jax.lax.dot_general(mf, e, (((1,), (1,)), ((), ())),
                                         preferred_element_type=jnp.float32,
                                         precision=jax.lax.Precision.HIGHEST)
        scnt[...] += jax.lax.dot_general(mf, jnp.ones_like(e),
                                         (((1,), (1,)), ((), ())),
                                         preferred_element_type=jnp.float32,
                                         precision=jax.lax.Precision.HIGHEST)

    @pl.when(ph == 1)
    def _phase_b():
        m = _onehot(b)
        mf = m.astype(jnp.float32)
        e = escr[pl.ds(step, 1), :]
        snode = jax.lax.dot_general(ssum[...], mf, (((0,), (0,)), ((), ())),
                                    preferred_element_type=jnp.float32,
                                    precision=jax.lax.Precision.HIGHEST)
        probs = e / snode
        score = jnp.log(probs) + gscr[pl.ds(step, 1), :]

        neg = jnp.float32(-jnp.inf)
        sm = jnp.where(m, score, neg)
        bmax = jnp.max(sm, axis=1, keepdims=True)
        bidx = jnp.min(jnp.where(sm == bmax, i, _BIG), axis=1, keepdims=True)
        sel = (i == bidx).astype(jnp.float32)
        pw = jax.lax.dot_general(sel, probs, (((1,), (1,)), ((), ())),
                                 preferred_element_type=jnp.float32,
                                 precision=jax.lax.Precision.HIGHEST)

        upd = bmax > smax[...]
        smax[...] = jnp.where(upd, bmax, smax[...])
        sidx[...] = jnp.where(upd, bidx, sidx[...])
        spwin[...] = jnp.where(upd, pw, spwin[...])

        @pl.when(step == _NB - 1)
        def _fin():
            r = jax.lax.broadcasted_iota(jnp.int32, (_G, _G), 0)
            c = jax.lax.broadcasted_iota(jnp.int32, (_G, _G), 1)
            tri = (c > r).astype(jnp.float32)
            starts = jax.lax.dot_general(tri, scnt[...],
                                         (((1,), (0,)), ((), ())),
                                         preferred_element_type=jnp.float32,
                                         precision=jax.lax.Precision.HIGHEST)
            p_ref[...] = spwin[...]
            a_ref[...] = sidx[...]
            act_ref[...] = sidx[...] - starts.astype(jnp.int32)


def kernel(X, batch, W0, b0, W1, b1, Wf, bf):
    bfull = batch.astype(jnp.int32).reshape(_NB, _B)
    b0r = b0.astype(jnp.float32).reshape(1, -1)
    b1r = b1.astype(jnp.float32).reshape(1, -1)
    wfr = Wf.astype(jnp.float32).reshape(-1, 1).T
    bfr = bf.astype(jnp.float32).reshape(1, 1)

    out_shapes = (
        jax.ShapeDtypeStruct((_G, 1), jnp.float32),
        jax.ShapeDtypeStruct((_G, 1), jnp.int32),
        jax.ShapeDtypeStruct((_G, 1), jnp.int32),
    )
    p, act, a = pl.pallas_call(
        _body,
        grid=(2, _NB),
        in_specs=[
            pl.BlockSpec((_B, 128),
                         lambda ph, s: (jnp.where(ph == 0, s, _NB - 1), 0)),
            pl.BlockSpec((_NB, _B), lambda ph, s: (0, 0)),
            pl.BlockSpec((128, 64), lambda ph, s: (0, 0)),
            pl.BlockSpec((1, 64), lambda ph, s: (0, 0)),
            pl.BlockSpec((64, 64), lambda ph, s: (0, 0)),
            pl.BlockSpec((1, 64), lambda ph, s: (0, 0)),
            pl.BlockSpec((1, 64), lambda ph, s: (0, 0)),
            pl.BlockSpec((1, 1), lambda ph, s: (0, 0)),
        ],
        out_specs=(
            pl.BlockSpec((_G, 1), lambda ph, s: (0, 0)),
            pl.BlockSpec((_G, 1), lambda ph, s: (0, 0)),
            pl.BlockSpec((_G, 1), lambda ph, s: (0, 0)),
        ),
        out_shape=out_shapes,
        scratch_shapes=[
            pltpu.VMEM((_NB, _B), jnp.float32),
            pltpu.VMEM((_NB, _B), jnp.float32),
            pltpu.VMEM((_G, 1), jnp.float32),
            pltpu.VMEM((_G, 1), jnp.float32),
            pltpu.VMEM((_G, 1), jnp.int32),
            pltpu.VMEM((_G, 1), jnp.float32),
            pltpu.VMEM((_G, 1), jnp.float32),
        ],
        compiler_params=pltpu.CompilerParams(
            dimension_semantics=("arbitrary", "arbitrary"),
        ),
    )(X, bfull, W0, b0r, W1, b1r, wfr, bfr)
    return (p[:, 0], act[:, 0], a[:, 0])


# speedup vs baseline: 1.3737x; 1.3737x over previous
import numpy as np
import jax
import jax.numpy as jnp
from jax.experimental import pallas as pl
from jax.experimental.pallas import tpu as pltpu

_N = 100000
_B = 4000
_NB = _N // _B
_G = 64
_BIG = np.int32(2**31 - 1)


def _threefry_bits(j):
    k0 = jnp.uint32(0)
    k1 = jnp.uint32(1234)
    ks2 = k0 ^ k1 ^ jnp.uint32(0x1BD11BDA)
    ks = [k0, k1, ks2]
    rots = [[13, 15, 26, 6], [17, 29, 16, 24]]
    x0 = jnp.zeros_like(j) + ks[0]
    x1 = j + ks[1]
    for i in range(5):
        for r in rots[i % 2]:
            x0 = x0 + x1
            x1 = (x1 << r) | (x1 >> (32 - r))
            x1 = x1 ^ x0
        x0 = x0 + ks[(i + 1) % 3]
        x1 = x1 + ks[(i + 2) % 3] + jnp.uint32(i + 1)
    return x0 ^ x1


def _gumbel_from_bits(bits):
    tiny = jnp.float32(np.finfo(np.float32).tiny)
    fb = (bits >> 9) | jnp.uint32(0x3F800000)
    f = jax.lax.bitcast_convert_type(fb, jnp.float32) - jnp.float32(1.0)
    u = jnp.maximum(tiny, f * (jnp.float32(1.0) - tiny) + tiny)
    return -jnp.log(-jnp.log(u))


def _onehot(b):
    gcol = 63 - jax.lax.broadcasted_iota(jnp.int32, (_G, 1), 0)
    return b == gcol


def _body(xb_ref, bt_ref, w0_ref, b0_ref, w1_ref, b1_ref, wf_ref,
          bf_ref, p_ref, act_ref, a_ref,
          gscr, escr, ssum, smax, sidx, spwin, scnt):
    ph = pl.program_id(0)
    step = pl.program_id(1)
    i = step * _B + jax.lax.broadcasted_iota(jnp.int32, (1, _B), 1)
    b = bt_ref[pl.ds(step, 1), :]

    @pl.when(ph == 0)
    def _phase_a():
        @pl.when(step == 0)
        def _init():
            ssum[...] = jnp.zeros_like(ssum)
            smax[...] = jnp.full_like(smax, -jnp.inf)
            sidx[...] = jnp.full_like(sidx, _BIG)
            spwin[...] = jnp.zeros_like(spwin)
            scnt[...] = jnp.zeros_like(scnt)
            for t in range(0, _NB, 8):
                rows = min(8, _NB - t)
                b8 = bt_ref[pl.ds(t, rows), :]
                i8 = (t * _B
                      + jax.lax.broadcasted_iota(jnp.int32, (rows, _B), 0) * _B
                      + jax.lax.broadcasted_iota(jnp.int32, (rows, _B), 1))
                j8 = ((63 - b8) * _N + i8).astype(jnp.uint32)
                gscr[pl.ds(t, rows), :] = _gumbel_from_bits(_threefry_bits(j8))

        xb = xb_ref[...]
        h = jax.lax.dot_general(xb, w0_ref[...], (((1,), (0,)), ((), ())),
                                preferred_element_type=jnp.float32)
        h = jnp.maximum(h + b0_ref[...], jnp.float32(0.0))
        h = jax.lax.dot_general(h, w1_ref[...], (((1,), (0,)), ((), ())),
                                preferred_element_type=jnp.float32)
        h = jnp.maximum(h + b1_ref[...], jnp.float32(0.0))
        logits = jax.lax.dot_general(wf_ref[...], h, (((1,), (1,)), ((), ())),
                                     preferred_element_type=jnp.float32)
        e = jnp.exp(logits + bf_ref[...])
        escr[pl.ds(step, 1), :] = e

        mf = _onehot(b).astype(jnp.float32)
        ssum[...] += jax.lax.dot_general(mf, e, (((1,), (1,)), ((), ())),
                                         preferred_element_type=jnp.float32,
                                         precision=jax.lax.Precision.HIGHEST)
        scnt[...] += jax.lax.dot_general(mf, jnp.ones_like(e),
                                         (((1,), (1,)), ((), ())),
                                         preferred_element_type=jnp.float32,
                                         precision=jax.lax.Precision.HIGHEST)

    @pl.when(ph == 1)
    def _phase_b():
        m = _onehot(b)
        mf = m.astype(jnp.float32)
        e = escr[pl.ds(step, 1), :]
        snode = jax.lax.dot_general(ssum[...], mf, (((0,), (0,)), ((), ())),
                                    preferred_element_type=jnp.float32,
                                    precision=jax.lax.Precision.HIGHEST)
        probs = e / snode
        score = jnp.log(probs) + gscr[pl.ds(step, 1), :]

        neg = jnp.float32(-jnp.inf)
        sm = jnp.where(m, score, neg)
        bmax = jnp.max(sm, axis=1, keepdims=True)
        bidx = jnp.min(jnp.where(sm == bmax, i, _BIG), axis=1, keepdims=True)
        sel = (i == bidx).astype(jnp.float32)
        pw = jax.lax.dot_general(sel, probs, (((1,), (1,)), ((), ())),
                                 preferred_element_type=jnp.float32,
                                 precision=jax.lax.Precision.HIGHEST)

        upd = bmax > smax[...]
        smax[...] = jnp.where(upd, bmax, smax[...])
        sidx[...] = jnp.where(upd, bidx, sidx[...])
        spwin[...] = jnp.where(upd, pw, spwin[...])

        @pl.when(step == _NB - 1)
        def _fin():
            r = jax.lax.broadcasted_iota(jnp.int32, (_G, _G), 0)
            c = jax.lax.broadcasted_iota(jnp.int32, (_G, _G), 1)
            tri = (c > r).astype(jnp.float32)
            starts = jax.lax.dot_general(tri, scnt[...],
                                         (((1,), (0,)), ((), ())),
                                         preferred_element_type=jnp.float32,
                                         precision=jax.lax.Precision.HIGHEST)
            p_ref[...] = spwin[...]
            a_ref[...] = sidx[...]
            act_ref[...] = sidx[...] - starts.astype(jnp.int32)


def kernel(X, batch, W0, b0, W1, b1, Wf, bf):
    bfull = batch.astype(jnp.int32).reshape(_NB, _B)
    b0r = b0.astype(jnp.float32).reshape(1, -1)
    b1r = b1.astype(jnp.float32).reshape(1, -1)
    wfr = Wf.astype(jnp.float32).reshape(-1, 1).T
    bfr = bf.astype(jnp.float32).reshape(1, 1)

    out_shapes = (
        jax.ShapeDtypeStruct((_G, 1), jnp.float32),
        jax.ShapeDtypeStruct((_G, 1), jnp.int32),
        jax.ShapeDtypeStruct((_G, 1), jnp.int32),
    )
    p, act, a = pl.pallas_call(
        _body,
        grid=(2, _NB),
        in_specs=[
            pl.BlockSpec((_B, 128),
                         lambda ph, s: (jnp.where(ph == 0, s, _NB - 1), 0)),
            pl.BlockSpec((_NB, _B), lambda ph, s: (0, 0)),
            pl.BlockSpec((128, 64), lambda ph, s: (0, 0)),
            pl.BlockSpec((1, 64), lambda ph, s: (0, 0)),
            pl.BlockSpec((64, 64), lambda ph, s: (0, 0)),
            pl.BlockSpec((1, 64), lambda ph, s: (0, 0)),
            pl.BlockSpec((1, 64), lambda ph, s: (0, 0)),
            pl.BlockSpec((1, 1), lambda ph, s: (0, 0)),
        ],
        out_specs=(
            pl.BlockSpec((_G, 1), lambda ph, s: (0, 0)),
            pl.BlockSpec((_G, 1), lambda ph, s: (0, 0)),
            pl.BlockSpec((_G, 1), lambda ph, s: (0, 0)),
        ),
        out_shape=out_shapes,
        scratch_shapes=[
            pltpu.VMEM((_NB, _B), jnp.float32),
            pltpu.VMEM((_NB, _B), jnp.float32),
            pltpu.VMEM((_G, 1), jnp.float32),
            pltpu.VMEM((_G, 1), jnp.float32),
            pltpu.VMEM((_G, 1), jnp.int32),
            pltpu.VMEM((_G, 1), jnp.float32),
            pltpu.VMEM((_G, 1), jnp.float32),
        ],
        compiler_params=pltpu.CompilerParams(
            dimension_semantics=("arbitrary", "arbitrary"),
        ),
    )(X, bfull, W0, b0r, W1, b1r, wfr, bfr)
    return (p[:, 0], act[:, 0], a[:, 0])


# speedup vs baseline: 1.6634x; 1.2108x over previous
import numpy as np
import jax
import jax.numpy as jnp
from jax.experimental import pallas as pl
from jax.experimental.pallas import tpu as pltpu

_N = 100000
_B = 10000
_NB = _N // _B
_G = 64
_BIG = np.int32(2**31 - 1)


def _threefry_bits(j):
    k0 = jnp.uint32(0)
    k1 = jnp.uint32(1234)
    ks2 = k0 ^ k1 ^ jnp.uint32(0x1BD11BDA)
    ks = [k0, k1, ks2]
    rots = [[13, 15, 26, 6], [17, 29, 16, 24]]
    x0 = jnp.zeros_like(j) + ks[0]
    x1 = j + ks[1]
    for i in range(5):
        for r in rots[i % 2]:
            x0 = x0 + x1
            x1 = (x1 << r) | (x1 >> (32 - r))
            x1 = x1 ^ x0
        x0 = x0 + ks[(i + 1) % 3]
        x1 = x1 + ks[(i + 2) % 3] + jnp.uint32(i + 1)
    return x0 ^ x1


def _gumbel_from_bits(bits):
    tiny = jnp.float32(np.finfo(np.float32).tiny)
    fb = (bits >> 9) | jnp.uint32(0x3F800000)
    f = jax.lax.bitcast_convert_type(fb, jnp.float32) - jnp.float32(1.0)
    u = jnp.maximum(tiny, f * (jnp.float32(1.0) - tiny) + tiny)
    return -jnp.log(-jnp.log(u))


def _onehot(b):
    gcol = 63 - jax.lax.broadcasted_iota(jnp.int32, (_G, 1), 0)
    return b == gcol


def _body(xb_ref, bt_ref, w0_ref, b0_ref, w1_ref, b1_ref, wf_ref,
          bf_ref, p_ref, act_ref, a_ref,
          gscr, escr, ssum, smax, sidx, spwin, scnt):
    ph = pl.program_id(0)
    step = pl.program_id(1)
    i = step * _B + jax.lax.broadcasted_iota(jnp.int32, (1, _B), 1)
    b = bt_ref[pl.ds(step, 1), :]

    @pl.when(ph == 0)
    def _phase_a():
        @pl.when(step == 0)
        def _init():
            ssum[...] = jnp.zeros_like(ssum)
            smax[...] = jnp.full_like(smax, -jnp.inf)
            sidx[...] = jnp.full_like(sidx, _BIG)
            spwin[...] = jnp.zeros_like(spwin)
            scnt[...] = jnp.zeros_like(scnt)
            for t in range(0, _NB, 8):
                rows = min(8, _NB - t)
                b8 = bt_ref[pl.ds(t, rows), :]
                i8 = (t * _B
                      + jax.lax.broadcasted_iota(jnp.int32, (rows, _B), 0) * _B
                      + jax.lax.broadcasted_iota(jnp.int32, (rows, _B), 1))
                j8 = ((63 - b8) * _N + i8).astype(jnp.uint32)
                gscr[pl.ds(t, rows), :] = _gumbel_from_bits(_threefry_bits(j8))

        xb = xb_ref[...]
        h = jax.lax.dot_general(xb, w0_ref[...], (((1,), (0,)), ((), ())),
                                preferred_element_type=jnp.float32)
        h = jnp.maximum(h + b0_ref[...], jnp.float32(0.0))
        h = jax.lax.dot_general(h, w1_ref[...], (((1,), (0,)), ((), ())),
                                preferred_element_type=jnp.float32)
        h = jnp.maximum(h + b1_ref[...], jnp.float32(0.0))
        logits = jax.lax.dot_general(wf_ref[...], h, (((1,), (1,)), ((), ())),
                                     preferred_element_type=jnp.float32)
        e = jnp.exp(logits + bf_ref[...])
        escr[pl.ds(step, 1), :] = e

        mf = _onehot(b).astype(jnp.float32)
        ssum[...] += jax.lax.dot_general(mf, e, (((1,), (1,)), ((), ())),
                                         preferred_element_type=jnp.float32,
                                         precision=jax.lax.Precision.HIGHEST)
        scnt[...] += jax.lax.dot_general(mf, jnp.ones_like(e),
                                         (((1,), (1,)), ((), ())),
                                         preferred_element_type=jnp.float32,
                                         precision=jax.lax.Precision.HIGHEST)

    @pl.when(ph == 1)
    def _phase_b():
        m = _onehot(b)
        mf = m.astype(jnp.float32)
        e = escr[pl.ds(step, 1), :]
        snode = jax.lax.dot_general(ssum[...], mf, (((0,), (0,)), ((), ())),
                                    preferred_element_type=jnp.float32,
                                    precision=jax.lax.Precision.HIGHEST)
        probs = e / snode
        score = jnp.log(probs) + gscr[pl.ds(step, 1), :]

        neg = jnp.float32(-jnp.inf)
        sm = jnp.where(m, score, neg)
        bmax = jnp.max(sm, axis=1, keepdims=True)
        bidx = jnp.min(jnp.where(sm == bmax, i, _BIG), axis=1, keepdims=True)
        sel = (i == bidx).astype(jnp.float32)
        pw = jax.lax.dot_general(sel, probs, (((1,), (1,)), ((), ())),
                                 preferred_element_type=jnp.float32,
                                 precision=jax.lax.Precision.HIGHEST)

        upd = bmax > smax[...]
        smax[...] = jnp.where(upd, bmax, smax[...])
        sidx[...] = jnp.where(upd, bidx, sidx[...])
        spwin[...] = jnp.where(upd, pw, spwin[...])

        @pl.when(step == _NB - 1)
        def _fin():
            r = jax.lax.broadcasted_iota(jnp.int32, (_G, _G), 0)
            c = jax.lax.broadcasted_iota(jnp.int32, (_G, _G), 1)
            tri = (c > r).astype(jnp.float32)
            starts = jax.lax.dot_general(tri, scnt[...],
                                         (((1,), (0,)), ((), ())),
                                         preferred_element_type=jnp.float32,
                                         precision=jax.lax.Precision.HIGHEST)
            p_ref[...] = spwin[...]
            a_ref[...] = sidx[...]
            act_ref[...] = sidx[...] - starts.astype(jnp.int32)


def kernel(X, batch, W0, b0, W1, b1, Wf, bf):
    bfull = batch.astype(jnp.int32).reshape(_NB, _B)
    b0r = b0.astype(jnp.float32).reshape(1, -1)
    b1r = b1.astype(jnp.float32).reshape(1, -1)
    wfr = Wf.astype(jnp.float32).reshape(-1, 1).T
    bfr = bf.astype(jnp.float32).reshape(1, 1)

    out_shapes = (
        jax.ShapeDtypeStruct((_G, 1), jnp.float32),
        jax.ShapeDtypeStruct((_G, 1), jnp.int32),
        jax.ShapeDtypeStruct((_G, 1), jnp.int32),
    )
    p, act, a = pl.pallas_call(
        _body,
        grid=(2, _NB),
        in_specs=[
            pl.BlockSpec((_B, 128),
                         lambda ph, s: (jnp.where(ph == 0, s, _NB - 1), 0)),
            pl.BlockSpec((_NB, _B), lambda ph, s: (0, 0)),
            pl.BlockSpec((128, 64), lambda ph, s: (0, 0)),
            pl.BlockSpec((1, 64), lambda ph, s: (0, 0)),
            pl.BlockSpec((64, 64), lambda ph, s: (0, 0)),
            pl.BlockSpec((1, 64), lambda ph, s: (0, 0)),
            pl.BlockSpec((1, 64), lambda ph, s: (0, 0)),
            pl.BlockSpec((1, 1), lambda ph, s: (0, 0)),
        ],
        out_specs=(
            pl.BlockSpec((_G, 1), lambda ph, s: (0, 0)),
            pl.BlockSpec((_G, 1), lambda ph, s: (0, 0)),
            pl.BlockSpec((_G, 1), lambda ph, s: (0, 0)),
        ),
        out_shape=out_shapes,
        scratch_shapes=[
            pltpu.VMEM((_NB, _B), jnp.float32),
            pltpu.VMEM((_NB, _B), jnp.float32),
            pltpu.VMEM((_G, 1), jnp.float32),
            pltpu.VMEM((_G, 1), jnp.float32),
            pltpu.VMEM((_G, 1), jnp.int32),
            pltpu.VMEM((_G, 1), jnp.float32),
            pltpu.VMEM((_G, 1), jnp.float32),
        ],
        compiler_params=pltpu.CompilerParams(
            dimension_semantics=("arbitrary", "arbitrary"),
        ),
    )(X, bfull, W0, b0r, W1, b1r, wfr, bfr)
    return (p[:, 0], act[:, 0], a[:, 0])


# speedup vs baseline: 1.7128x; 1.0297x over previous
import numpy as np
import jax
import jax.numpy as jnp
from jax.experimental import pallas as pl
from jax.experimental.pallas import tpu as pltpu

_N = 100000
_B = 20000
_NB = _N // _B
_G = 64
_BIG = np.int32(2**31 - 1)


def _threefry_bits(j):
    k0 = jnp.uint32(0)
    k1 = jnp.uint32(1234)
    ks2 = k0 ^ k1 ^ jnp.uint32(0x1BD11BDA)
    ks = [k0, k1, ks2]
    rots = [[13, 15, 26, 6], [17, 29, 16, 24]]
    x0 = jnp.zeros_like(j) + ks[0]
    x1 = j + ks[1]
    for i in range(5):
        for r in rots[i % 2]:
            x0 = x0 + x1
            x1 = (x1 << r) | (x1 >> (32 - r))
            x1 = x1 ^ x0
        x0 = x0 + ks[(i + 1) % 3]
        x1 = x1 + ks[(i + 2) % 3] + jnp.uint32(i + 1)
    return x0 ^ x1


def _gumbel_from_bits(bits):
    tiny = jnp.float32(np.finfo(np.float32).tiny)
    fb = (bits >> 9) | jnp.uint32(0x3F800000)
    f = jax.lax.bitcast_convert_type(fb, jnp.float32) - jnp.float32(1.0)
    u = jnp.maximum(tiny, f * (jnp.float32(1.0) - tiny) + tiny)
    return -jnp.log(-jnp.log(u))


def _onehot(b):
    gcol = 63 - jax.lax.broadcasted_iota(jnp.int32, (_G, 1), 0)
    return b == gcol


def _body(xb_ref, bt_ref, w0_ref, b0_ref, w1_ref, b1_ref, wf_ref,
          bf_ref, p_ref, act_ref, a_ref,
          gscr, escr, ssum, smax, sidx, spwin, scnt):
    ph = pl.program_id(0)
    step = pl.program_id(1)
    i = step * _B + jax.lax.broadcasted_iota(jnp.int32, (1, _B), 1)
    b = bt_ref[pl.ds(step, 1), :]

    @pl.when(ph == 0)
    def _phase_a():
        @pl.when(step == 0)
        def _init():
            ssum[...] = jnp.zeros_like(ssum)
            smax[...] = jnp.full_like(smax, -jnp.inf)
            sidx[...] = jnp.full_like(sidx, _BIG)
            spwin[...] = jnp.zeros_like(spwin)
            scnt[...] = jnp.zeros_like(scnt)
            for t in range(0, _NB, 8):
                rows = min(8, _NB - t)
                b8 = bt_ref[pl.ds(t, rows), :]
                i8 = (t * _B
                      + jax.lax.broadcasted_iota(jnp.int32, (rows, _B), 0) * _B
                      + jax.lax.broadcasted_iota(jnp.int32, (rows, _B), 1))
                j8 = ((63 - b8) * _N + i8).astype(jnp.uint32)
                gscr[pl.ds(t, rows), :] = _gumbel_from_bits(_threefry_bits(j8))

        xb = xb_ref[...]
        h = jax.lax.dot_general(xb, w0_ref[...], (((1,), (0,)), ((), ())),
                                preferred_element_type=jnp.float32)
        h = jnp.maximum(h + b0_ref[...], jnp.float32(0.0))
        h = jax.lax.dot_general(h, w1_ref[...], (((1,), (0,)), ((), ())),
                                preferred_element_type=jnp.float32)
        h = jnp.maximum(h + b1_ref[...], jnp.float32(0.0))
        logits = jax.lax.dot_general(wf_ref[...], h, (((1,), (1,)), ((), ())),
                                     preferred_element_type=jnp.float32)
        e = jnp.exp(logits + bf_ref[...])
        escr[pl.ds(step, 1), :] = e

        mf = _onehot(b).astype(jnp.float32)
        ssum[...] += jax.lax.dot_general(mf, e, (((1,), (1,)), ((), ())),
                                         preferred_element_type=jnp.float32,
                                         precision=jax.lax.Precision.HIGHEST)
        scnt[...] += jax.lax.dot_general(mf, jnp.ones_like(e),
                                         (((1,), (1,)), ((), ())),
                                         preferred_element_type=jnp.float32,
                                         precision=jax.lax.Precision.HIGHEST)

    @pl.when(ph == 1)
    def _phase_b():
        m = _onehot(b)
        mf = m.astype(jnp.float32)
        e = escr[pl.ds(step, 1), :]
        snode = jax.lax.dot_general(ssum[...], mf, (((0,), (0,)), ((), ())),
                                    preferred_element_type=jnp.float32,
                                    precision=jax.lax.Precision.HIGHEST)
        probs = e / snode
        score = jnp.log(probs) + gscr[pl.ds(step, 1), :]

        neg = jnp.float32(-jnp.inf)
        sm = jnp.where(m, score, neg)
        bmax = jnp.max(sm, axis=1, keepdims=True)
        bidx = jnp.min(jnp.where(sm == bmax, i, _BIG), axis=1, keepdims=True)
        sel = (i == bidx).astype(jnp.float32)
        pw = jax.lax.dot_general(sel, probs, (((1,), (1,)), ((), ())),
                                 preferred_element_type=jnp.float32,
                                 precision=jax.lax.Precision.HIGHEST)

        upd = bmax > smax[...]
        smax[...] = jnp.where(upd, bmax, smax[...])
        sidx[...] = jnp.where(upd, bidx, sidx[...])
        spwin[...] = jnp.where(upd, pw, spwin[...])

        @pl.when(step == _NB - 1)
        def _fin():
            r = jax.lax.broadcasted_iota(jnp.int32, (_G, _G), 0)
            c = jax.lax.broadcasted_iota(jnp.int32, (_G, _G), 1)
            tri = (c > r).astype(jnp.float32)
            starts = jax.lax.dot_general(tri, scnt[...],
                                         (((1,), (0,)), ((), ())),
                                         preferred_element_type=jnp.float32,
                                         precision=jax.lax.Precision.HIGHEST)
            p_ref[...] = spwin[...]
            a_ref[...] = sidx[...]
            act_ref[...] = sidx[...] - starts.astype(jnp.int32)


def kernel(X, batch, W0, b0, W1, b1, Wf, bf):
    bfull = batch.astype(jnp.int32).reshape(_NB, _B)
    b0r = b0.astype(jnp.float32).reshape(1, -1)
    b1r = b1.astype(jnp.float32).reshape(1, -1)
    wfr = Wf.astype(jnp.float32).reshape(-1, 1).T
    bfr = bf.astype(jnp.float32).reshape(1, 1)

    out_shapes = (
        jax.ShapeDtypeStruct((_G, 1), jnp.float32),
        jax.ShapeDtypeStruct((_G, 1), jnp.int32),
        jax.ShapeDtypeStruct((_G, 1), jnp.int32),
    )
    p, act, a = pl.pallas_call(
        _body,
        grid=(2, _NB),
        in_specs=[
            pl.BlockSpec((_B, 128),
                         lambda ph, s: (jnp.where(ph == 0, s, _NB - 1), 0)),
            pl.BlockSpec((_NB, _B), lambda ph, s: (0, 0)),
            pl.BlockSpec((128, 64), lambda ph, s: (0, 0)),
            pl.BlockSpec((1, 64), lambda ph, s: (0, 0)),
            pl.BlockSpec((64, 64), lambda ph, s: (0, 0)),
            pl.BlockSpec((1, 64), lambda ph, s: (0, 0)),
            pl.BlockSpec((1, 64), lambda ph, s: (0, 0)),
            pl.BlockSpec((1, 1), lambda ph, s: (0, 0)),
        ],
        out_specs=(
            pl.BlockSpec((_G, 1), lambda ph, s: (0, 0)),
            pl.BlockSpec((_G, 1), lambda ph, s: (0, 0)),
            pl.BlockSpec((_G, 1), lambda ph, s: (0, 0)),
        ),
        out_shape=out_shapes,
        scratch_shapes=[
            pltpu.VMEM((_NB, _B), jnp.float32),
            pltpu.VMEM((_NB, _B), jnp.float32),
            pltpu.VMEM((_G, 1), jnp.float32),
            pltpu.VMEM((_G, 1), jnp.float32),
            pltpu.VMEM((_G, 1), jnp.int32),
            pltpu.VMEM((_G, 1), jnp.float32),
            pltpu.VMEM((_G, 1), jnp.float32),
        ],
        compiler_params=pltpu.CompilerParams(
            dimension_semantics=("arbitrary", "arbitrary"),
        ),
    )(X, bfull, W0, b0r, W1, b1r, wfr, bfr)
    return (p[:, 0], act[:, 0], a[:, 0])
